# Initial kernel scaffold; baseline (speedup 1.0000x reference)
import jax
import jax.numpy as jnp
from jax.experimental import pallas as pl


def kernel(x, targets, anchors, img_size):
    raise NotImplementedError("write your pallas kernel here")


# speedup vs baseline: 3.9571x; 3.9571x over previous
import functools

import jax
import jax.numpy as jnp
from jax import lax
from jax.experimental import pallas as pl
from jax.experimental.pallas import tpu as pltpu


def _bce_pos(s):
    return -jnp.maximum(jnp.log(s), -100.0)


def _bce_neg(s):
    return -jnp.maximum(jnp.log(1.0 - s), -100.0)


def _sigmoid(x):
    return 1.0 / (1.0 + jnp.exp(-x))


def _yolo_kernel(x_ref, t_ref, anch_ref, stride_ref, pred_ref, gath_ref,
                 nsum_ref, loss_ref, *, B, A, G, NO, NT):
    b = pl.program_id(0)
    a = pl.program_id(1)
    NC = NO - 5
    cells = G * G
    stride = stride_ref[0, 0]

    xa = x_ref[0, 0]
    s_all = _sigmoid(xa)
    e23 = jnp.exp(xa[2:4, :])

    cell_i = lax.broadcasted_iota(jnp.int32, (1, cells), 1)
    mx = (cell_i // G).astype(jnp.float32)
    my = (cell_i % G).astype(jnp.float32)
    grid01 = jnp.concatenate([mx, my], axis=0)
    aw_a = anch_ref[a, 0] / stride
    ah_a = anch_ref[a, 1] / stride
    row01 = (s_all[0:2, :] + grid01) * stride
    row2 = (e23[0:1, :] * aw_a) * stride
    row3 = (e23[1:2, :] * ah_a) * stride
    pred_blk = jnp.concatenate([row01, row2, row3, s_all[4:, :]], axis=0)
    pred_ref[0, 0] = pred_blk.T

    nb = jnp.sum(_bce_neg(s_all[4:5, :]))

    timg = t_ref[:, 0:1].astype(jnp.int32)
    b01 = t_ref[:, 2:4] / stride
    b23 = t_ref[:, 4:6] / stride
    bwh = b23 - b01
    cxy = b01 + bwh / 2.0
    gij = cxy.astype(jnp.int32)
    gij = jnp.where(gij[:, 0:1] < 0, 0, gij)
    gij = jnp.where(gij[:, 1:2] < 0, 0, gij)
    gij = jnp.where(gij[:, 0:1] >= G, G - 1, gij)
    gij = jnp.where(gij[:, 1:2] >= G, G - 1, gij)
    gx = gij[:, 0:1]
    gy = gij[:, 1:2]
    cell_t = gy * G + gx

    ci = lax.broadcasted_iota(jnp.int32, (NT, cells), 1)
    hot = jnp.where((ci == cell_t) & (timg == b), 1.0, 0.0)
    contrib = lax.dot_general(
        hot, xa, (((1,), (1,)), ((), ())),
        precision=lax.Precision.HIGHEST,
        preferred_element_type=jnp.float32)

    first = jnp.logical_and(b == 0, a == 0)

    @pl.when(first)
    def _():
        gath_ref[:] = jnp.zeros_like(gath_ref)
        nsum_ref[0, 0] = 0.0

    gath_ref[pl.ds(a, 1)] += contrib[None]
    nsum_ref[0, 0] += nb

    io_t = lax.broadcasted_iota(jnp.int32, (NT, NT), 0)
    io_s = lax.broadcasted_iota(jnp.int32, (NT, NT), 1)
    diag = io_t == io_s

    def _row(col_f32):
        m = jnp.where(diag, jnp.broadcast_to(col_f32, (NT, NT)), 0.0)
        return jnp.sum(m, axis=0, keepdims=True)

    @pl.when(jnp.logical_and(b == B - 1, a == A - 1))
    def _():
        tcls = t_ref[:, 1:2].astype(jnp.int32)
        bw = bwh[:, 0:1]
        bh = bwh[:, 1:2]
        iou_c = []
        for aa in range(A):
            aw = anch_ref[aa, 0] / stride
            ah = anch_ref[aa, 1] / stride
            inter = jnp.minimum(aw, bw) * jnp.minimum(ah, bh)
            iou_c.append(inter / (aw * ah + bw * bh - inter))
        best = jnp.where(iou_c[1] > iou_c[0], 1, 0)
        m01 = jnp.maximum(iou_c[0], iou_c[1])
        best = jnp.where(iou_c[2] > m01, 2, best)

        key = (timg * A + best) * cells + cell_t
        key_r = _row(key.astype(jnp.float32))
        keq = key.astype(jnp.float32) == key_r
        loser = jnp.any(jnp.logical_and(keq, io_s > io_t),
                        axis=1, keepdims=True)
        w = jnp.where(loser, 0.0, 1.0)
        n_obj = jnp.maximum(jnp.sum(w), 1.0)

        sel = jnp.zeros((NT, NO), jnp.float32)
        anch_tw = jnp.zeros((NT, 1), jnp.float32)
        anch_th = jnp.zeros((NT, 1), jnp.float32)
        for aa in range(A):
            m_a = best == aa
            sel = sel + jnp.where(m_a, gath_ref[aa], 0.0)
            anch_tw = anch_tw + jnp.where(m_a, anch_ref[aa, 0] / stride, 0.0)
            anch_th = anch_th + jnp.where(m_a, anch_ref[aa, 1] / stride, 0.0)

        txy = cxy - jnp.floor(cxy)
        twh_w = jnp.log(bw / anch_tw + 1e-10)
        twh_h = jnp.log(bh / anch_th + 1e-10)

        p01 = _sigmoid(sel[:, 0:2])
        lx = jnp.sum(w * (p01[:, 0:1] - txy[:, 0:1]) ** 2)
        ly = jnp.sum(w * (p01[:, 1:2] - txy[:, 1:2]) ** 2)
        lw_ = jnp.sum(w * (sel[:, 2:3] - twh_w) ** 2)
        lh = jnp.sum(w * (sel[:, 3:4] - twh_h) ** 2)
        p4 = _sigmoid(sel[:, 4:5])
        lobj = jnp.sum(w * _bce_pos(p4))

        pc = _sigmoid(sel[:, 5:])
        cls_i = lax.broadcasted_iota(jnp.int32, (NT, NC), 1)
        oh = jnp.where(cls_i == tcls, 1.0, 0.0)
        bce_c = oh * _bce_pos(pc) + (1.0 - oh) * _bce_neg(pc)
        lcls = jnp.sum(w * bce_c)

        act = [jnp.logical_or(best == aa, iou_c[aa] > 0.5) for aa in range(A)]
        key2 = [((timg * A + aa) * cells + cell_t).astype(jnp.float32)
                for aa in range(A)]
        act_r = [_row(jnp.where(act[aa], 1.0, 0.0)) for aa in range(A)]
        key2_r = [_row(key2[aa]) for aa in range(A)]
        corr = jnp.float32(0.0)
        count = jnp.float32(0.0)
        for aa in range(A):
            dup = jnp.zeros((NT, 1), jnp.bool_)
            for bb in range(A):
                keq2 = key2[aa] == key2_r[bb]
                if bb < aa:
                    order = io_s <= io_t
                else:
                    order = io_s < io_t
                hit = jnp.logical_and(jnp.logical_and(keq2, order),
                                      act_r[bb] > 0.5)
                dup = jnp.logical_or(dup, jnp.any(hit, axis=1, keepdims=True))
            fa = jnp.where(jnp.logical_and(act[aa], jnp.logical_not(dup)),
                           1.0, 0.0)
            count = count + jnp.sum(fa)
            s4a = _sigmoid(gath_ref[aa][:, 4:5])
            corr = corr + jnp.sum(fa * _bce_neg(s4a))
        total = jnp.float32(B * A * cells)
        n_noobj = jnp.maximum(total - count, 1.0)
        lnoobj = (nsum_ref[0, 0] - corr) / n_noobj

        loss = ((lx + ly + lw_ + lh + lobj) / n_obj
                + 100.0 * lnoobj
                + lcls / (n_obj * NC))
        loss_ref[0, 0] = loss


def kernel(x, targets, anchors, img_size):
    B, C, W, H = x.shape
    A = anchors.shape[0]
    NO = C // A
    NT = targets.shape[0]
    G = W
    stride_f = jnp.asarray(img_size // W, jnp.float32).reshape(1, 1)
    x4 = x.reshape(B, A, NO, G * G)

    grid = (B, A)
    out_shapes = (
        jax.ShapeDtypeStruct((B, A, G * G, NO), jnp.float32),
        jax.ShapeDtypeStruct((A, NT, NO), jnp.float32),
        jax.ShapeDtypeStruct((1, 1), jnp.float32),
        jax.ShapeDtypeStruct((1, 1), jnp.float32),
    )
    in_specs = [
        pl.BlockSpec((1, 1, NO, G * G), lambda b, a: (b, a, 0, 0)),
        pl.BlockSpec((NT, 6), lambda b, a: (0, 0)),
        pl.BlockSpec(memory_space=pltpu.SMEM),
        pl.BlockSpec(memory_space=pltpu.SMEM),
    ]
    out_specs = (
        pl.BlockSpec((1, 1, G * G, NO), lambda b, a: (b, a, 0, 0)),
        pl.BlockSpec((A, NT, NO), lambda b, a: (0, 0, 0)),
        pl.BlockSpec(memory_space=pltpu.SMEM),
        pl.BlockSpec(memory_space=pltpu.SMEM),
    )
    body = functools.partial(_yolo_kernel, B=B, A=A, G=G, NO=NO, NT=NT)
    pred4, _, _, loss = pl.pallas_call(
        body,
        grid=grid,
        in_specs=in_specs,
        out_specs=out_specs,
        out_shape=out_shapes,
    )(x4, targets, anchors, stride_f)
    return pred4.reshape(B, A * G * G, NO), loss[0, 0]


# speedup vs baseline: 5.1081x; 1.2909x over previous
import functools

import jax
import jax.numpy as jnp
from jax import lax
from jax.experimental import pallas as pl
from jax.experimental.pallas import tpu as pltpu


def _bce_pos(s):
    return -jnp.maximum(jnp.log(s), -100.0)


def _bce_neg(s):
    return -jnp.maximum(jnp.log(1.0 - s), -100.0)


def _sigmoid(x):
    return 1.0 / (1.0 + jnp.exp(-x))


def _yolo_kernel(x_ref, t_ref, anch_ref, stride_ref, pred_ref, gath_ref,
                 nsum_ref, loss_ref, *, B, A, G, NO, NT):
    b = pl.program_id(0)
    NC = NO - 5
    cells = G * G
    stride = stride_ref[0, 0]

    cell_i = lax.broadcasted_iota(jnp.int32, (1, cells), 1)
    mx = (cell_i // G).astype(jnp.float32)
    my = (cell_i % G).astype(jnp.float32)
    grid01 = jnp.concatenate([mx, my], axis=0)
    nb = jnp.float32(0.0)
    for aa in range(A):
        xa = x_ref[0, aa]
        s_a = _sigmoid(xa)
        e2 = jnp.exp(xa[2:3, :])
        e3 = jnp.exp(xa[3:4, :])
        row01 = (s_a[0:2, :] + grid01) * stride
        row2 = (e2 * (anch_ref[aa, 0] / stride)) * stride
        row3 = (e3 * (anch_ref[aa, 1] / stride)) * stride
        blk = jnp.concatenate([row01, row2, row3, s_a[4:, :]], axis=0)
        pred_ref[0, aa] = blk.T
        nb = nb + jnp.sum(_bce_neg(s_a[4:5, :]))

    timg = t_ref[0:1, :].astype(jnp.int32)
    bx1 = t_ref[2:3, :] / stride
    by1 = t_ref[3:4, :] / stride
    bx2 = t_ref[4:5, :] / stride
    by2 = t_ref[5:6, :] / stride
    bw = bx2 - bx1
    bh = by2 - by1
    cxx = bx1 + bw / 2.0
    cyy = by1 + bh / 2.0
    gx = cxx.astype(jnp.int32)
    gy = cyy.astype(jnp.int32)
    m = gx < 0
    gx = jnp.where(m, 0, gx)
    gy = jnp.where(m, 0, gy)
    m = gy < 0
    gx = jnp.where(m, 0, gx)
    gy = jnp.where(m, 0, gy)
    m = gx >= G
    gx = jnp.where(m, G - 1, gx)
    gy = jnp.where(m, G - 1, gy)
    m = gy >= G
    gx = jnp.where(m, G - 1, gx)
    gy = jnp.where(m, G - 1, gy)
    cell_t = gy * G + gx

    x2 = jnp.reshape(x_ref[0], (A * NO, cells))
    ci = lax.broadcasted_iota(jnp.int32, (cells, NT), 0)
    hot = jnp.where((ci == cell_t) & (timg == b), 1.0, 0.0)
    contrib = lax.dot_general(
        x2, hot, (((1,), (0,)), ((), ())),
        precision=lax.Precision.HIGHEST,
        preferred_element_type=jnp.float32)

    @pl.when(b == 0)
    def _():
        gath_ref[:] = jnp.zeros_like(gath_ref)
        nsum_ref[0, 0] = 0.0

    for aa in range(A):
        gath_ref[aa] += contrib[aa * NO:(aa + 1) * NO, :]
    nsum_ref[0, 0] += nb

    @pl.when(b == B - 1)
    def _():
        io_t = lax.broadcasted_iota(jnp.int32, (NT, NT), 0)
        io_s = lax.broadcasted_iota(jnp.int32, (NT, NT), 1)
        diag = io_t == io_s

        def _col(row_f32):
            mm = jnp.where(diag, jnp.broadcast_to(row_f32, (NT, NT)), 0.0)
            return jnp.sum(mm, axis=1, keepdims=True)

        tcls = t_ref[1:2, :].astype(jnp.int32)
        iou_c = []
        for aa in range(A):
            aw = anch_ref[aa, 0] / stride
            ah = anch_ref[aa, 1] / stride
            inter = jnp.minimum(aw, bw) * jnp.minimum(ah, bh)
            iou_c.append(inter / (aw * ah + bw * bh - inter))
        best = jnp.where(iou_c[1] > iou_c[0], 1, 0)
        m01 = jnp.maximum(iou_c[0], iou_c[1])
        best = jnp.where(iou_c[2] > m01, 2, best)

        key = ((timg * A + best) * cells + cell_t).astype(jnp.float32)
        keq = _col(key) == key
        loser = jnp.any(jnp.logical_and(keq, io_t > io_s),
                        axis=0, keepdims=True)
        w = jnp.where(loser, 0.0, 1.0)
        n_obj = jnp.maximum(jnp.sum(w), 1.0)

        sel = jnp.zeros((NO, NT), jnp.float32)
        anch_tw = jnp.zeros((1, NT), jnp.float32)
        anch_th = jnp.zeros((1, NT), jnp.float32)
        for aa in range(A):
            m_a = best == aa
            sel = sel + jnp.where(m_a, gath_ref[aa], 0.0)
            anch_tw = anch_tw + jnp.where(m_a, anch_ref[aa, 0] / stride, 0.0)
            anch_th = anch_th + jnp.where(m_a, anch_ref[aa, 1] / stride, 0.0)

        tx = cxx - jnp.floor(cxx)
        ty = cyy - jnp.floor(cyy)
        twx = jnp.log(bw / anch_tw + 1e-10)
        twy = jnp.log(bh / anch_th + 1e-10)

        lx = jnp.sum(w * (_sigmoid(sel[0:1, :]) - tx) ** 2)
        ly = jnp.sum(w * (_sigmoid(sel[1:2, :]) - ty) ** 2)
        lw_ = jnp.sum(w * (sel[2:3, :] - twx) ** 2)
        lh = jnp.sum(w * (sel[3:4, :] - twy) ** 2)
        lobj = jnp.sum(w * _bce_pos(_sigmoid(sel[4:5, :])))

        pc = _sigmoid(sel[5:, :])
        cls_i = lax.broadcasted_iota(jnp.int32, (NC, NT), 0)
        oh = jnp.where(cls_i == tcls, 1.0, 0.0)
        bce_c = oh * _bce_pos(pc) + (1.0 - oh) * _bce_neg(pc)
        lcls = jnp.sum(w * bce_c)

        act = []
        key2m = []
        for aa in range(A):
            a_act = jnp.logical_or(best == aa, iou_c[aa] > 0.5)
            k2 = ((timg * A + aa) * cells + cell_t).astype(jnp.float32)
            act.append(a_act)
            key2m.append(jnp.where(a_act, k2, -1.0))
        key2_col = [_col(k) for k in key2m]
        corr = jnp.float32(0.0)
        count = jnp.float32(0.0)
        for aa in range(A):
            dup = jnp.zeros((1, NT), jnp.bool_)
            for bb in range(A):
                keq2 = key2_col[bb] == key2m[aa]
                if bb < aa:
                    order = io_t <= io_s
                else:
                    order = io_t < io_s
                dup = jnp.logical_or(
                    dup, jnp.any(jnp.logical_and(keq2, order),
                                 axis=0, keepdims=True))
            fa = jnp.where(jnp.logical_and(act[aa], jnp.logical_not(dup)),
                           1.0, 0.0)
            count = count + jnp.sum(fa)
            corr = corr + jnp.sum(fa * _bce_neg(_sigmoid(gath_ref[aa][4:5, :])))
        total = jnp.float32(B * A * cells)
        n_noobj = jnp.maximum(total - count, 1.0)
        lnoobj = (nsum_ref[0, 0] - corr) / n_noobj

        loss = ((lx + ly + lw_ + lh + lobj) / n_obj
                + 100.0 * lnoobj
                + lcls / (n_obj * NC))
        loss_ref[0, 0] = loss


def kernel(x, targets, anchors, img_size):
    B, C, W, H = x.shape
    A = anchors.shape[0]
    NO = C // A
    NT = targets.shape[0]
    G = W
    stride_f = jnp.asarray(img_size // W, jnp.float32).reshape(1, 1)
    x4 = x.reshape(B, A, NO, G * G)
    t_t = targets.T

    grid = (B,)
    out_shapes = (
        jax.ShapeDtypeStruct((B, A, G * G, NO), jnp.float32),
        jax.ShapeDtypeStruct((A, NO, NT), jnp.float32),
        jax.ShapeDtypeStruct((1, 1), jnp.float32),
        jax.ShapeDtypeStruct((1, 1), jnp.float32),
    )
    in_specs = [
        pl.BlockSpec((1, A, NO, G * G), lambda b: (b, 0, 0, 0)),
        pl.BlockSpec((6, NT), lambda b: (0, 0)),
        pl.BlockSpec(memory_space=pltpu.SMEM),
        pl.BlockSpec(memory_space=pltpu.SMEM),
    ]
    out_specs = (
        pl.BlockSpec((1, A, G * G, NO), lambda b: (b, 0, 0, 0)),
        pl.BlockSpec((A, NO, NT), lambda b: (0, 0, 0)),
        pl.BlockSpec(memory_space=pltpu.SMEM),
        pl.BlockSpec(memory_space=pltpu.SMEM),
    )
    body = functools.partial(_yolo_kernel, B=B, A=A, G=G, NO=NO, NT=NT)
    pred4, _, _, loss = pl.pallas_call(
        body,
        grid=grid,
        in_specs=in_specs,
        out_specs=out_specs,
        out_shape=out_shapes,
    )(x4, t_t, anchors, stride_f)
    return pred4.reshape(B, A * G * G, NO), loss[0, 0]


# speedup vs baseline: 6.8738x; 1.3457x over previous
import functools

import jax
import jax.numpy as jnp
from jax import lax
from jax.experimental import pallas as pl
from jax.experimental.pallas import tpu as pltpu


def _bce_pos(s):
    return -jnp.maximum(jnp.log(s), -100.0)


def _bce_neg(s):
    return -jnp.maximum(jnp.log(1.0 - s), -100.0)


def _sigmoid(x):
    return 1.0 / (1.0 + jnp.exp(-x))


def _yolo_kernel(x_ref, t_ref, anch_ref, stride_ref, pred_ref, gath_ref,
                 nsum_ref, loss_ref, hot_ref, *, B, A, G, NO, NT):
    b = pl.program_id(0)
    NC = NO - 5
    cells = G * G
    stride = stride_ref[0, 0]

    timg = t_ref[0:1, :].astype(jnp.int32)
    bx1 = t_ref[2:3, :] / stride
    by1 = t_ref[3:4, :] / stride
    bx2 = t_ref[4:5, :] / stride
    by2 = t_ref[5:6, :] / stride
    bw = bx2 - bx1
    bh = by2 - by1
    cxx = bx1 + bw / 2.0
    cyy = by1 + bh / 2.0
    gx = cxx.astype(jnp.int32)
    gy = cyy.astype(jnp.int32)
    m = gx < 0
    gx = jnp.where(m, 0, gx)
    gy = jnp.where(m, 0, gy)
    m = gy < 0
    gx = jnp.where(m, 0, gx)
    gy = jnp.where(m, 0, gy)
    m = gx >= G
    gx = jnp.where(m, G - 1, gx)
    gy = jnp.where(m, G - 1, gy)
    m = gy >= G
    gx = jnp.where(m, G - 1, gx)
    gy = jnp.where(m, G - 1, gy)
    cell_t = gy * G + gx

    @pl.when(b == 0)
    def _():
        ci = lax.broadcasted_iota(jnp.int32, (cells, NT), 0)
        hot_ref[:] = jnp.where(ci == cell_t, 1.0, 0.0)
        gath_ref[:] = jnp.zeros_like(gath_ref)
        nsum_ref[0, 0] = 0.0

    cell_i = lax.broadcasted_iota(jnp.int32, (1, cells), 1)
    mx = (cell_i // G).astype(jnp.float32)
    my = (cell_i % G).astype(jnp.float32)
    grid01 = jnp.concatenate([mx, my], axis=0)
    nb = jnp.float32(0.0)
    hot = hot_ref[:]
    imgm = timg == b
    blks = []
    for aa in range(A):
        xa = x_ref[0, aa]
        s_a = _sigmoid(xa)
        e2 = jnp.exp(xa[2:3, :])
        e3 = jnp.exp(xa[3:4, :])
        row01 = (s_a[0:2, :] + grid01) * stride
        row2 = (e2 * (anch_ref[aa, 0] / stride)) * stride
        row3 = (e3 * (anch_ref[aa, 1] / stride)) * stride
        blk = jnp.concatenate([row01, row2, row3, s_a[4:, :]], axis=0)
        blks.append(blk.T)
        nb = nb + jnp.sum(_bce_neg(s_a[4:5, :]))
        contrib = lax.dot_general(
            xa, hot, (((1,), (0,)), ((), ())),
            precision=lax.Precision.DEFAULT,
            preferred_element_type=jnp.float32)
        gath_ref[aa] += jnp.where(imgm, contrib, 0.0)
    pred_ref[0] = jnp.concatenate(blks, axis=0)
    nsum_ref[0, 0] += nb

    @pl.when(b == B - 1)
    def _():
        io_t = lax.broadcasted_iota(jnp.int32, (NT, NT), 0)
        io_s = lax.broadcasted_iota(jnp.int32, (NT, NT), 1)
        diag = io_t == io_s

        def _col(row_f32):
            mm = jnp.where(diag, jnp.broadcast_to(row_f32, (NT, NT)), 0.0)
            return jnp.sum(mm, axis=1, keepdims=True)

        tcls = t_ref[1:2, :].astype(jnp.int32)
        iou_c = []
        for aa in range(A):
            aw = anch_ref[aa, 0] / stride
            ah = anch_ref[aa, 1] / stride
            inter = jnp.minimum(aw, bw) * jnp.minimum(ah, bh)
            iou_c.append(inter / (aw * ah + bw * bh - inter))
        best = jnp.where(iou_c[1] > iou_c[0], 1, 0)
        m01 = jnp.maximum(iou_c[0], iou_c[1])
        best = jnp.where(iou_c[2] > m01, 2, best)

        key = ((timg * A + best) * cells + cell_t).astype(jnp.float32)
        keq = _col(key) == key
        loser = jnp.any(jnp.logical_and(keq, io_t > io_s),
                        axis=0, keepdims=True)
        w = jnp.where(loser, 0.0, 1.0)
        n_obj = jnp.maximum(jnp.sum(w), 1.0)

        sel = jnp.zeros((NO, NT), jnp.float32)
        anch_tw = jnp.zeros((1, NT), jnp.float32)
        anch_th = jnp.zeros((1, NT), jnp.float32)
        for aa in range(A):
            m_a = best == aa
            sel = sel + jnp.where(m_a, gath_ref[aa], 0.0)
            anch_tw = anch_tw + jnp.where(m_a, anch_ref[aa, 0] / stride, 0.0)
            anch_th = anch_th + jnp.where(m_a, anch_ref[aa, 1] / stride, 0.0)

        tx = cxx - jnp.floor(cxx)
        ty = cyy - jnp.floor(cyy)
        twx = jnp.log(bw / anch_tw + 1e-10)
        twy = jnp.log(bh / anch_th + 1e-10)

        lx = jnp.sum(w * (_sigmoid(sel[0:1, :]) - tx) ** 2)
        ly = jnp.sum(w * (_sigmoid(sel[1:2, :]) - ty) ** 2)
        lw_ = jnp.sum(w * (sel[2:3, :] - twx) ** 2)
        lh = jnp.sum(w * (sel[3:4, :] - twy) ** 2)
        lobj = jnp.sum(w * _bce_pos(_sigmoid(sel[4:5, :])))

        pc = _sigmoid(sel[5:, :])
        cls_i = lax.broadcasted_iota(jnp.int32, (NC, NT), 0)
        oh = jnp.where(cls_i == tcls, 1.0, 0.0)
        bce_c = oh * _bce_pos(pc) + (1.0 - oh) * _bce_neg(pc)
        lcls = jnp.sum(w * bce_c)

        act = []
        key2m = []
        for aa in range(A):
            a_act = jnp.logical_or(best == aa, iou_c[aa] > 0.5)
            k2 = ((timg * A + aa) * cells + cell_t).astype(jnp.float32)
            act.append(a_act)
            key2m.append(jnp.where(a_act, k2, -1.0))
        key2_col = [_col(k) for k in key2m]
        corr = jnp.float32(0.0)
        count = jnp.float32(0.0)
        for aa in range(A):
            dup = jnp.zeros((1, NT), jnp.bool_)
            for bb in range(A):
                keq2 = key2_col[bb] == key2m[aa]
                if bb < aa:
                    order = io_t <= io_s
                else:
                    order = io_t < io_s
                dup = jnp.logical_or(
                    dup, jnp.any(jnp.logical_and(keq2, order),
                                 axis=0, keepdims=True))
            fa = jnp.where(jnp.logical_and(act[aa], jnp.logical_not(dup)),
                           1.0, 0.0)
            count = count + jnp.sum(fa)
            corr = corr + jnp.sum(fa * _bce_neg(_sigmoid(gath_ref[aa][4:5, :])))
        total = jnp.float32(B * A * cells)
        n_noobj = jnp.maximum(total - count, 1.0)
        lnoobj = (nsum_ref[0, 0] - corr) / n_noobj

        loss = ((lx + ly + lw_ + lh + lobj) / n_obj
                + 100.0 * lnoobj
                + lcls / (n_obj * NC))
        loss_ref[0, 0] = loss


def kernel(x, targets, anchors, img_size):
    B, C, W, H = x.shape
    A = anchors.shape[0]
    NO = C // A
    NT = targets.shape[0]
    G = W
    stride_f = jnp.asarray(img_size // W, jnp.float32).reshape(1, 1)
    x4 = x.reshape(B, A, NO, G * G)
    t_t = targets.T

    grid = (B,)
    out_shapes = (
        jax.ShapeDtypeStruct((B, A * G * G, NO), jnp.float32),
        jax.ShapeDtypeStruct((A, NO, NT), jnp.float32),
        jax.ShapeDtypeStruct((1, 1), jnp.float32),
        jax.ShapeDtypeStruct((1, 1), jnp.float32),
    )
    in_specs = [
        pl.BlockSpec((1, A, NO, G * G), lambda b: (b, 0, 0, 0)),
        pl.BlockSpec((6, NT), lambda b: (0, 0)),
        pl.BlockSpec(memory_space=pltpu.SMEM),
        pl.BlockSpec(memory_space=pltpu.SMEM),
    ]
    out_specs = (
        pl.BlockSpec((1, A * G * G, NO), lambda b: (b, 0, 0)),
        pl.BlockSpec((A, NO, NT), lambda b: (0, 0, 0)),
        pl.BlockSpec(memory_space=pltpu.SMEM),
        pl.BlockSpec(memory_space=pltpu.SMEM),
    )
    body = functools.partial(_yolo_kernel, B=B, A=A, G=G, NO=NO, NT=NT)
    pred, _, _, loss = pl.pallas_call(
        body,
        grid=grid,
        in_specs=in_specs,
        out_specs=out_specs,
        out_shape=out_shapes,
        scratch_shapes=[pltpu.VMEM((G * G, NT), jnp.float32)],
    )(x4, t_t, anchors, stride_f)
    return pred, loss[0, 0]
